# baseline (speedup 1.0000x reference)
import functools

import jax
import jax.numpy as jnp
from jax import lax
from jax.experimental import pallas as pl
from jax.experimental.pallas import tpu as pltpu
from jax.experimental.pallas import tpu_sc as plsc

D = 64
B = 16384
LMBDA = 0.0001

NC = 2
NS = 16
NW = NC * NS
BPW = B // NW


@functools.cache
def _sc_gather():
    mesh = plsc.VectorSubcoreMesh(core_axis_name="c", subcore_axis_name="s")
    out_t = [jax.ShapeDtypeStruct((B, D), jnp.float32)] * 6
    scratch = [
        pltpu.SMEM((BPW,), jnp.int32),
        pltpu.VMEM_SHARED((B,), jnp.int32),
        pltpu.VMEM((BPW, D), jnp.float32),
        pltpu.SemaphoreType.DMA,
    ]

    @functools.partial(pl.kernel, mesh=mesh, out_type=out_t,
                       scratch_types=scratch)
    def k(ph, pt, pr, nh, nt, nr, ent, rel,
          o_ph, o_pt, o_pr, o_nh, o_nt, o_nr,
          idx_s, idx_sh, rows, sem):
        wid = lax.axis_index("s") * NC + lax.axis_index("c")
        base = wid * BPW
        pairs = [(ph, ent, o_ph), (pt, ent, o_pt), (pr, rel, o_pr),
                 (nh, ent, o_nh), (nt, ent, o_nt), (nr, rel, o_nr)]
        for idx_hbm, table, out in pairs:
            pltpu.sync_copy(idx_hbm.at[pl.ds(base, BPW)],
                            idx_sh.at[pl.ds(base, BPW)])
            pltpu.sync_copy(idx_sh.at[pl.ds(base, BPW)], idx_s)

            def fire(i, _, table=table):
                off = idx_s[i]
                pltpu.make_async_copy(
                    table.at[pl.ds(off, 1)], rows.at[pl.ds(i, 1)], sem
                ).start()
                return 0

            def drain(i, _, table=table):
                pltpu.make_async_copy(
                    table.at[pl.ds(0, 1)], rows.at[pl.ds(i, 1)], sem
                ).wait()
                return 0

            lax.fori_loop(0, BPW, fire, 0)
            lax.fori_loop(0, BPW, drain, 0)
            pltpu.sync_copy(rows, out.at[pl.ds(base, BPW)])

    return k


def _tc_loss(ph, pt, pr, nh, nt, nr):
    BLK = 2048

    def body(ph_ref, pt_ref, pr_ref, nh_ref, nt_ref, nr_ref, out_ref):
        @pl.when(pl.program_id(0) == 0)
        def _():
            out_ref[0, 0] = 0.0

        phv, ptv, prv = ph_ref[...], pt_ref[...], pr_ref[...]
        nhv, ntv, nrv = nh_ref[...], nt_ref[...], nr_ref[...]
        p = jnp.sum(phv * prv * ptv, axis=-1)
        n = jnp.sum(nhv * nrv * ntv, axis=-1)
        lf = jnp.sum(jax.nn.softplus(-p) + jax.nn.softplus(n))
        rg = jnp.sum(phv * phv + ptv * ptv + prv * prv
                     + nhv * nhv + ntv * ntv + nrv * nrv)
        out_ref[0, 0] += lf + LMBDA * rg

    spec = pl.BlockSpec((BLK, D), lambda i: (i, 0))
    out = pl.pallas_call(
        body,
        grid=(B // BLK,),
        in_specs=[spec] * 6,
        out_specs=pl.BlockSpec(memory_space=pltpu.SMEM),
        out_shape=jax.ShapeDtypeStruct((1, 1), jnp.float32),
    )(ph, pt, pr, nh, nt, nr)
    return out[0, 0]


def kernel(pos_h, pos_t, pos_r, neg_h, neg_t, neg_r,
           ent_embeddings, rel_embeddings):
    idxs = [x.astype(jnp.int32) for x in
            (pos_h, pos_t, pos_r, neg_h, neg_t, neg_r)]
    ph, pt, pr, nh, nt, nr = _sc_gather()(
        *idxs, ent_embeddings, rel_embeddings)
    return _tc_loss(ph, pt, pr, nh, nt, nr)
